# baseline (speedup 1.0000x reference)
import functools

import jax
import jax.numpy as jnp
from jax import lax
from jax.experimental import pallas as pl
from jax.experimental.pallas import tpu as pltpu
from jax.experimental.pallas import tpu_sc as plsc

NUM_TOKENS = 1000000
EMBED_DIM = 64
BATCH = 4096
SEQ = 200

NC = 2
NS = 16
NW = NC * NS

B_TOTAL = BATCH * SEQ
PER_W = B_TOTAL // NW
IDX_CHUNK = 128
N_STEPS = PER_W // IDX_CHUNK


@jax.jit
def _embed_gather(token_ids_flat, emb_table):
    mesh = plsc.VectorSubcoreMesh(core_axis_name="c", subcore_axis_name="s")

    idx3 = token_ids_flat.reshape(NW, N_STEPS, IDX_CHUNK)

    @functools.partial(
        pl.kernel,
        mesh=mesh,
        compiler_params=pltpu.CompilerParams(use_tc_tiling_on_sc=False),
        out_type=jax.ShapeDtypeStruct((B_TOTAL, EMBED_DIM), jnp.float32),
        scratch_types=[
            pltpu.VMEM((N_STEPS, IDX_CHUNK), jnp.int32),
            pltpu.VMEM((IDX_CHUNK, EMBED_DIM), jnp.float32),
            pltpu.SemaphoreType.DMA,
        ],
    )
    def k(idx_hbm, table_hbm, out_hbm, idx_v, rows_v, gsem):
        wid = lax.axis_index("s") * NC + lax.axis_index("c")
        base = wid * PER_W
        pltpu.sync_copy(idx_hbm.at[wid], idx_v)

        def step(j, _):
            pltpu.async_copy(table_hbm.at[idx_v.at[j]], rows_v, gsem).wait()
            pltpu.sync_copy(
                rows_v, out_hbm.at[pl.ds(base + j * IDX_CHUNK, IDX_CHUNK)]
            )
            return 0

        lax.fori_loop(0, N_STEPS, step, 0)

    return k(idx3, emb_table)


def kernel(token_ids, emb_table):
    out = _embed_gather(token_ids.reshape(-1), emb_table)
    return out.reshape(BATCH, SEQ, EMBED_DIM)


# speedup vs baseline: 1.1131x; 1.1131x over previous
import functools

import jax
import jax.numpy as jnp
from jax import lax
from jax.experimental import pallas as pl
from jax.experimental.pallas import tpu as pltpu
from jax.experimental.pallas import tpu_sc as plsc

NUM_TOKENS = 1000000
EMBED_DIM = 64
BATCH = 4096
SEQ = 200

NC = 2
NS = 16
NW = NC * NS

B_TOTAL = BATCH * SEQ
PER_W = B_TOTAL // NW
CHUNK = 512
N_STEPS = PER_W // CHUNK
N_PAIRS = N_STEPS // 2


@jax.jit
def _embed_gather(token_ids_flat, emb_table):
    mesh = plsc.VectorSubcoreMesh(core_axis_name="c", subcore_axis_name="s")

    idx3 = token_ids_flat.reshape(NW, N_STEPS, CHUNK)

    @functools.partial(
        pl.kernel,
        mesh=mesh,
        compiler_params=pltpu.CompilerParams(use_tc_tiling_on_sc=False),
        out_type=jax.ShapeDtypeStruct((B_TOTAL, EMBED_DIM), jnp.float32),
        scratch_types=[
            pltpu.VMEM((N_STEPS, CHUNK), jnp.int32),
            pltpu.VMEM((CHUNK, EMBED_DIM), jnp.float32),
            pltpu.VMEM((CHUNK, EMBED_DIM), jnp.float32),
            pltpu.SemaphoreType.DMA,
            pltpu.SemaphoreType.DMA,
            pltpu.SemaphoreType.DMA,
            pltpu.SemaphoreType.DMA,
        ],
    )
    def k(idx_hbm, table_hbm, out_hbm, idx_v, buf0, buf1, g0, g1, w0, w1):
        wid = lax.axis_index("s") * NC + lax.axis_index("c")
        base = wid * PER_W
        pltpu.sync_copy(idx_hbm.at[wid], idx_v)

        def gather(j, buf, sem):
            pltpu.async_copy(table_hbm.at[idx_v.at[j]], buf, sem)

        def gather_wait(j, buf, sem):
            pltpu.make_async_copy(table_hbm.at[idx_v.at[j]], buf, sem).wait()

        def write(j, buf, sem):
            pltpu.async_copy(
                buf, out_hbm.at[pl.ds(base + j * CHUNK, CHUNK)], sem
            )

        def write_wait(j, buf, sem):
            pltpu.make_async_copy(
                buf, out_hbm.at[pl.ds(base + j * CHUNK, CHUNK)], sem
            ).wait()

        gather(0, buf0, g0)

        def body(p, _):
            s0 = 2 * p
            gather_wait(s0, buf0, g0)
            pl.when(p > 0)(lambda: write_wait(s0 - 1, buf1, w1))
            gather(s0 + 1, buf1, g1)
            write(s0, buf0, w0)
            gather_wait(s0 + 1, buf1, g1)
            write_wait(s0, buf0, w0)
            pl.when(p + 1 < N_PAIRS)(lambda: gather(s0 + 2, buf0, g0))
            write(s0 + 1, buf1, w1)
            return 0

        lax.fori_loop(0, N_PAIRS, body, 0)
        write_wait(N_STEPS - 1, buf1, w1)

    return k(idx3, emb_table)


def kernel(token_ids, emb_table):
    out = _embed_gather(token_ids.reshape(-1), emb_table)
    return out.reshape(BATCH, SEQ, EMBED_DIM)
